# baseline (speedup 1.0000x reference)
import jax
import jax.numpy as jnp
from jax import lax
from jax.experimental import pallas as pl
from jax.experimental.pallas import tpu as pltpu

_KF = 196
_KEEP = 147
_NI = 588
_IKEEP = 49
_D = 1024
_LNF = 4


def _merge_schedule():
    n, sched = _NI, []
    while n > _IKEEP:
        m = min(n // 3, n - _IKEEP)
        if m <= 0:
            break
        sched.append(m)
        n -= m
    return sched


_SCHED = tuple(_merge_schedule())

_PREC_DECIDE = lax.Precision.DEFAULT
_PREC_MLP = lax.Precision.DEFAULT


def _split3(v):
    f32 = jnp.float32
    bf16 = jnp.bfloat16
    hi = v.astype(bf16)
    r1 = v - hi.astype(f32)
    mid = r1.astype(bf16)
    lo = (r1 - mid.astype(f32)).astype(bf16)
    return jnp.concatenate([hi, mid, lo], axis=0)


def _onehot_matmul(m, stacked):
    f32 = jnp.float32
    m3 = jnp.concatenate([m, m, m], axis=1)
    return lax.dot_general(m3, stacked, (((1,), (0,)), ((), ())),
                           preferred_element_type=f32)


def _row_from_col(col, n):
    return jnp.swapaxes(col, 0, 1)


def _col_from_row(row, n):
    return jnp.swapaxes(row, 0, 1)


def _norm_rows(v):
    n = jnp.sqrt(jnp.sum(v * v, axis=1, keepdims=True))
    return v / jnp.maximum(n, 1e-12)


_CPB = 1


def _body(gate_ref, x_ref, w1_ref, b1_ref, w2_ref, b2_ref, out_ref):
    for c in range(_CPB):
        _one_clip(c, gate_ref, x_ref, w1_ref, b1_ref, w2_ref, b2_ref, out_ref)


def _one_clip(c, gate_ref, x_ref, w1_ref, b1_ref, w2_ref, b2_ref, out_ref):
    f32 = jnp.float32
    xb = x_ref[c]
    kf = xb[:_KF, :]
    incr = xb[_KF:, :]

    kf_n = _norm_rows(kf)
    sim_kf = lax.dot_general(kf_n, kf_n, (((1,), (1,)), ((), ())),
                             precision=_PREC_DECIDE, preferred_element_type=f32)
    red_col = jnp.sum(sim_kf, axis=1, keepdims=True) - 1.0
    red_row = _row_from_col(red_col, _KF)
    jj = lax.broadcasted_iota(jnp.int32, (_KF, _KF), 0)
    ii = lax.broadcasted_iota(jnp.int32, (_KF, _KF), 1)
    ones_kf = jnp.full((1, _KF), 1.0, dtype=f32)
    ahead = ((red_col < red_row) | ((red_col == red_row) & (jj < ii))).astype(f32)
    rank = lax.dot_general(ones_kf, ahead, (((1,), (0,)), ((), ())),
                           preferred_element_type=f32)
    keep_row = rank < float(_KEEP)
    keepf = keep_row.astype(f32)

    trif = (jj < ii).astype(f32)
    pos = lax.dot_general(keepf, trif, (((1,), (0,)), ((), ())),
                          preferred_element_type=f32)
    sel_iota = lax.broadcasted_iota(jnp.int32, (_KEEP, _KF), 0).astype(f32)
    sel = jnp.where((pos == sel_iota) & keep_row, 1.0, 0.0).astype(jnp.bfloat16)

    kf_parts = _split3(kf)
    ksum = _onehot_matmul(sel, kf_parts)

    incr_parts = _split3(incr)
    incr_n = _norm_rows(incr)
    kcnt = jnp.full((_KEEP, 1), 1.0, dtype=f32)
    alive_row = jnp.full((1, _NI), 1.0, dtype=f32)
    ones_ni = jnp.full((1, _NI), 1.0, dtype=jnp.bfloat16)

    kidx = lax.broadcasted_iota(jnp.int32, (_KEEP, _NI), 0)
    jj6 = lax.broadcasted_iota(jnp.int32, (_NI, _NI), 0)
    ii6 = lax.broadcasted_iota(jnp.int32, (_NI, _NI), 1)
    idxlt6 = jj6 < ii6

    for m in _SCHED:
        kavg = ksum / kcnt
        kavg_n = _norm_rows(kavg)
        sim = lax.dot_general(kavg_n, incr_n, (((1,), (1,)), ((), ())),
                              precision=_PREC_DECIDE,
                              preferred_element_type=f32)
        smax_row = jnp.max(sim, axis=0, keepdims=True)
        best = jnp.min(jnp.where(sim == smax_row, kidx, _KEEP),
                       axis=0, keepdims=True)
        s_row = jnp.where(alive_row > 0.5, smax_row, -3.0)
        s_col = _col_from_row(s_row, _NI)
        ahead6 = ((s_col > s_row) |
                  ((s_col == s_row) & idxlt6)).astype(jnp.bfloat16)
        rank6 = lax.dot_general(ones_ni, ahead6, (((1,), (0,)), ((), ())),
                                preferred_element_type=f32)
        merged = (alive_row > 0.5) & (rank6 < float(m))
        mmat = jnp.where((kidx == best) & merged, 1.0, 0.0).astype(jnp.bfloat16)
        ksum = ksum + _onehot_matmul(mmat, incr_parts)
        kcnt = kcnt + jnp.sum(mmat.astype(f32), axis=1, keepdims=True)
        alive_row = alive_row * (1.0 - merged.astype(f32))

    out_kf = ksum / kcnt
    tri6 = idxlt6.astype(f32)
    posi = lax.dot_general(alive_row, tri6, (((1,), (0,)), ((), ())),
                           preferred_element_type=f32)
    pio = lax.broadcasted_iota(jnp.int32, (_IKEEP, _NI), 0).astype(f32)
    perm = jnp.where((posi == pio) & (alive_row > 0.5), 1.0, 0.0).astype(jnp.bfloat16)
    out_incr = _onehot_matmul(perm, incr_parts)
    xc = jnp.concatenate([out_kf, out_incr], axis=0)
    xc = xc * gate_ref[0, 0]

    h = lax.dot_general(xc, w1_ref[...], (((1,), (0,)), ((), ())),
                        precision=_PREC_MLP, preferred_element_type=f32)
    h = h + b1_ref[...]
    h = 0.5 * h * (1.0 + lax.erf(h * 0.7071067811865476))
    out = lax.dot_general(h, w2_ref[...], (((1,), (0,)), ((), ())),
                          precision=_PREC_MLP, preferred_element_type=f32)
    out = out + b2_ref[...]
    out_ref[c] = out


def kernel(x, W1, b1, W2, b2, compress=1, local_num_frames=4):
    assert x.shape[1] == _KF
    n_clips = x.shape[0] // _LNF
    xr = x.reshape(n_clips, _LNF * _KF, x.shape[-1])
    gate = (jnp.asarray(compress, jnp.float32)
            * (jnp.asarray(local_num_frames, jnp.float32)
               / jnp.asarray(local_num_frames, jnp.float32))).reshape(1, 1)
    b1r = b1.reshape(1, -1)
    b2r = b2.reshape(1, -1)

    out = pl.pallas_call(
        _body,
        grid=(n_clips // _CPB,),
        in_specs=[
            pl.BlockSpec((1, 1), lambda i: (0, 0)),
            pl.BlockSpec((_CPB, _LNF * _KF, _D), lambda i: (i, 0, 0)),
            pl.BlockSpec(W1.shape, lambda i: (0, 0)),
            pl.BlockSpec((1, b1.shape[0]), lambda i: (0, 0)),
            pl.BlockSpec(W2.shape, lambda i: (0, 0)),
            pl.BlockSpec((1, b2.shape[0]), lambda i: (0, 0)),
        ],
        out_specs=pl.BlockSpec((_CPB, _KF, W2.shape[1]), lambda i: (i, 0, 0)),
        out_shape=jax.ShapeDtypeStruct((n_clips, _KF, W2.shape[1]),
                                       jnp.float32),
        compiler_params=pltpu.CompilerParams(
            dimension_semantics=("parallel",),
        ),
    )(gate, xr, W1, b1r, W2, b2r)
    return out


# speedup vs baseline: 1.1160x; 1.1160x over previous
import jax
import jax.numpy as jnp
from jax import lax
from jax.experimental import pallas as pl
from jax.experimental.pallas import tpu as pltpu

_KF = 196
_KEEP = 147
_NI = 588
_IKEEP = 49
_D = 1024
_LNF = 4


def _merge_schedule():
    n, sched = _NI, []
    while n > _IKEEP:
        m = min(n // 3, n - _IKEEP)
        if m <= 0:
            break
        sched.append(m)
        n -= m
    return sched


_SCHED = tuple(_merge_schedule())

_PREC_DECIDE = lax.Precision.DEFAULT
_PREC_MLP = lax.Precision.DEFAULT


def _split3(v):
    f32 = jnp.float32
    bf16 = jnp.bfloat16
    hi = v.astype(bf16)
    r1 = v - hi.astype(f32)
    mid = r1.astype(bf16)
    lo = (r1 - mid.astype(f32)).astype(bf16)
    return jnp.concatenate([hi, mid, lo], axis=0)


def _onehot_matmul(m, stacked):
    f32 = jnp.float32
    m3 = jnp.concatenate([m, m, m], axis=1)
    return lax.dot_general(m3, stacked, (((1,), (0,)), ((), ())),
                           preferred_element_type=f32)


def _row_from_col(col, n):
    return jnp.swapaxes(col, 0, 1)


def _col_from_row(row, n):
    return jnp.swapaxes(row, 0, 1)


def _norm_rows(v):
    n = jnp.sqrt(jnp.sum(v * v, axis=1, keepdims=True))
    return v / jnp.maximum(n, 1e-12)


_CPB = 1


def _body(gate_ref, x_ref, w1_ref, b1_ref, w2_ref, b2_ref, out_ref):
    for c in range(_CPB):
        _one_clip(c, gate_ref, x_ref, w1_ref, b1_ref, w2_ref, b2_ref, out_ref)


def _one_clip(c, gate_ref, x_ref, w1_ref, b1_ref, w2_ref, b2_ref, out_ref):
    f32 = jnp.float32
    kf = x_ref[4 * c]
    incr = jnp.concatenate([x_ref[4 * c + 1], x_ref[4 * c + 2],
                            x_ref[4 * c + 3]], axis=0)

    kf_n = _norm_rows(kf)
    sim_kf = lax.dot_general(kf_n, kf_n, (((1,), (1,)), ((), ())),
                             precision=_PREC_DECIDE, preferred_element_type=f32)
    red_col = jnp.sum(sim_kf, axis=1, keepdims=True) - 1.0
    red_row = _row_from_col(red_col, _KF)
    jj = lax.broadcasted_iota(jnp.int32, (_KF, _KF), 0)
    ii = lax.broadcasted_iota(jnp.int32, (_KF, _KF), 1)
    ones_kf = jnp.full((1, _KF), 1.0, dtype=f32)
    ahead = ((red_col < red_row) | ((red_col == red_row) & (jj < ii))).astype(f32)
    rank = lax.dot_general(ones_kf, ahead, (((1,), (0,)), ((), ())),
                           preferred_element_type=f32)
    keep_row = rank < float(_KEEP)
    keepf = keep_row.astype(f32)

    trif = (jj < ii).astype(f32)
    pos = lax.dot_general(keepf, trif, (((1,), (0,)), ((), ())),
                          preferred_element_type=f32)
    sel_iota = lax.broadcasted_iota(jnp.int32, (_KEEP, _KF), 0).astype(f32)
    sel = jnp.where((pos == sel_iota) & keep_row, 1.0, 0.0).astype(jnp.bfloat16)

    kf_parts = _split3(kf)
    ksum = _onehot_matmul(sel, kf_parts)

    incr_parts = _split3(incr)
    incr_n = _norm_rows(incr)
    kcnt = jnp.full((_KEEP, 1), 1.0, dtype=f32)
    alive_row = jnp.full((1, _NI), 1.0, dtype=f32)
    ones_ni = jnp.full((1, _NI), 1.0, dtype=jnp.bfloat16)

    kidx = lax.broadcasted_iota(jnp.int32, (_KEEP, _NI), 0)
    jj6 = lax.broadcasted_iota(jnp.int32, (_NI, _NI), 0)
    ii6 = lax.broadcasted_iota(jnp.int32, (_NI, _NI), 1)
    idxlt6 = jj6 < ii6

    for m in _SCHED:
        kavg = ksum / kcnt
        kavg_n = _norm_rows(kavg)
        sim = lax.dot_general(kavg_n, incr_n, (((1,), (1,)), ((), ())),
                              precision=_PREC_DECIDE,
                              preferred_element_type=f32)
        smax_row = jnp.max(sim, axis=0, keepdims=True)
        best = jnp.min(jnp.where(sim == smax_row, kidx, _KEEP),
                       axis=0, keepdims=True)
        s_row = jnp.where(alive_row > 0.5, smax_row, -3.0)
        s_col = _col_from_row(s_row, _NI)
        ahead6 = ((s_col > s_row) |
                  ((s_col == s_row) & idxlt6)).astype(jnp.bfloat16)
        rank6 = lax.dot_general(ones_ni, ahead6, (((1,), (0,)), ((), ())),
                                preferred_element_type=f32)
        merged = (alive_row > 0.5) & (rank6 < float(m))
        mmat = jnp.where((kidx == best) & merged, 1.0, 0.0).astype(jnp.bfloat16)
        ksum = ksum + _onehot_matmul(mmat, incr_parts)
        kcnt = kcnt + jnp.sum(mmat.astype(f32), axis=1, keepdims=True)
        alive_row = alive_row * (1.0 - merged.astype(f32))

    out_kf = ksum / kcnt
    tri6 = idxlt6.astype(f32)
    posi = lax.dot_general(alive_row, tri6, (((1,), (0,)), ((), ())),
                           preferred_element_type=f32)
    pio = lax.broadcasted_iota(jnp.int32, (_IKEEP, _NI), 0).astype(f32)
    perm = jnp.where((posi == pio) & (alive_row > 0.5), 1.0, 0.0).astype(jnp.bfloat16)
    out_incr = _onehot_matmul(perm, incr_parts)
    xc = jnp.concatenate([out_kf, out_incr], axis=0)
    xc = xc * gate_ref[0, 0]

    h = lax.dot_general(xc, w1_ref[...], (((1,), (0,)), ((), ())),
                        precision=_PREC_MLP, preferred_element_type=f32)
    h = h + b1_ref[...]
    h = 0.5 * h * (1.0 + lax.erf(h * 0.7071067811865476))
    out = lax.dot_general(h, w2_ref[...], (((1,), (0,)), ((), ())),
                          precision=_PREC_MLP, preferred_element_type=f32)
    out = out + b2_ref[...]
    out_ref[c] = out


def kernel(x, W1, b1, W2, b2, compress=1, local_num_frames=4):
    assert x.shape[1] == _KF
    n_clips = x.shape[0] // _LNF
    gate = (jnp.asarray(compress, jnp.float32)
            * (jnp.asarray(local_num_frames, jnp.float32)
               / jnp.asarray(local_num_frames, jnp.float32))).reshape(1, 1)
    b1r = b1.reshape(1, -1)
    b2r = b2.reshape(1, -1)

    out = pl.pallas_call(
        _body,
        grid=(n_clips // _CPB,),
        in_specs=[
            pl.BlockSpec((1, 1), lambda i: (0, 0)),
            pl.BlockSpec((_CPB * _LNF, _KF, _D), lambda i: (i, 0, 0)),
            pl.BlockSpec(W1.shape, lambda i: (0, 0)),
            pl.BlockSpec((1, b1.shape[0]), lambda i: (0, 0)),
            pl.BlockSpec(W2.shape, lambda i: (0, 0)),
            pl.BlockSpec((1, b2.shape[0]), lambda i: (0, 0)),
        ],
        out_specs=pl.BlockSpec((_CPB, _KF, W2.shape[1]), lambda i: (i, 0, 0)),
        out_shape=jax.ShapeDtypeStruct((n_clips, _KF, W2.shape[1]),
                                       jnp.float32),
        compiler_params=pltpu.CompilerParams(
            dimension_semantics=("parallel",),
        ),
    )(gate, x, W1, b1r, W2, b2r)
    return out


# speedup vs baseline: 1.1598x; 1.0392x over previous
import jax
import jax.numpy as jnp
from jax import lax
from jax.experimental import pallas as pl
from jax.experimental.pallas import tpu as pltpu

_KF = 196
_KEEP = 147
_NI = 588
_IKEEP = 49
_D = 1024
_LNF = 4


def _merge_schedule():
    n, sched = _NI, []
    while n > _IKEEP:
        m = min(n // 3, n - _IKEEP)
        if m <= 0:
            break
        sched.append(m)
        n -= m
    return sched


_SCHED = tuple(_merge_schedule())

_PREC_DECIDE = lax.Precision.DEFAULT
_PREC_MLP = lax.Precision.DEFAULT


def _split3(v):
    f32 = jnp.float32
    bf16 = jnp.bfloat16
    hi = v.astype(bf16)
    r1 = v - hi.astype(f32)
    mid = r1.astype(bf16)
    lo = (r1 - mid.astype(f32)).astype(bf16)
    return jnp.concatenate([hi, mid, lo], axis=0)


def _onehot_matmul(m, stacked):
    f32 = jnp.float32
    m3 = jnp.concatenate([m, m, m], axis=1)
    return lax.dot_general(m3, stacked, (((1,), (0,)), ((), ())),
                           preferred_element_type=f32)


def _row_from_col(col, n):
    return jnp.swapaxes(col, 0, 1)


def _col_from_row(row, n):
    return jnp.swapaxes(row, 0, 1)


def _norm_rows(v):
    n = jnp.sqrt(jnp.sum(v * v, axis=1, keepdims=True))
    return v / jnp.maximum(n, 1e-12)


_CPB = 1


def _body(gate_ref, x_ref, w1_ref, b1_ref, w2_ref, b2_ref, out_ref):
    for c in range(_CPB):
        _one_clip(c, gate_ref, x_ref, w1_ref, b1_ref, w2_ref, b2_ref, out_ref)


def _one_clip(c, gate_ref, x_ref, w1_ref, b1_ref, w2_ref, b2_ref, out_ref):
    f32 = jnp.float32
    kf = x_ref[4 * c]
    incr = jnp.concatenate([x_ref[4 * c + 1], x_ref[4 * c + 2],
                            x_ref[4 * c + 3]], axis=0)

    kf_n = _norm_rows(kf)
    sim_kf = lax.dot_general(kf_n, kf_n, (((1,), (1,)), ((), ())),
                             precision=_PREC_DECIDE, preferred_element_type=f32)
    red_col = jnp.sum(sim_kf, axis=1, keepdims=True) - 1.0
    red_row = _row_from_col(red_col, _KF)
    jj = lax.broadcasted_iota(jnp.int32, (_KF, _KF), 0)
    ii = lax.broadcasted_iota(jnp.int32, (_KF, _KF), 1)
    ahead = (red_col < red_row) | ((red_col == red_row) & (jj < ii))
    rank = jnp.sum(ahead.astype(jnp.int32), axis=0, keepdims=True)
    keep_row = rank < _KEEP
    keepf = keep_row.astype(f32)

    trif = (jj < ii).astype(f32)
    pos = lax.dot_general(keepf, trif, (((1,), (0,)), ((), ())),
                          preferred_element_type=f32)
    sel_iota = lax.broadcasted_iota(jnp.int32, (_KEEP, _KF), 0).astype(f32)
    sel = jnp.where((pos == sel_iota) & keep_row, 1.0, 0.0).astype(jnp.bfloat16)

    kf_parts = _split3(kf)
    ksum = _onehot_matmul(sel, kf_parts)

    incr_parts = _split3(incr)
    incr_n = _norm_rows(incr)
    kcnt = jnp.full((_KEEP, 1), 1.0, dtype=f32)
    alive_row = jnp.full((1, _NI), 1.0, dtype=f32)

    kidx = lax.broadcasted_iota(jnp.int32, (_KEEP, _NI), 0)
    jj6 = lax.broadcasted_iota(jnp.int32, (_NI, _NI), 0)
    ii6 = lax.broadcasted_iota(jnp.int32, (_NI, _NI), 1)
    idxlt6 = jj6 < ii6

    for m in _SCHED:
        kavg = ksum / kcnt
        kavg_n = _norm_rows(kavg)
        sim = lax.dot_general(kavg_n, incr_n, (((1,), (1,)), ((), ())),
                              precision=_PREC_DECIDE,
                              preferred_element_type=f32)
        smax_row = jnp.max(sim, axis=0, keepdims=True)
        best = jnp.min(jnp.where(sim == smax_row, kidx, _KEEP),
                       axis=0, keepdims=True)
        s_row = jnp.where(alive_row > 0.5, smax_row, -3.0)
        s_col = _col_from_row(s_row, _NI)
        ahead6 = (s_col > s_row) | ((s_col == s_row) & idxlt6)
        rank6 = jnp.sum(ahead6.astype(jnp.int32), axis=0,
                        keepdims=True)
        merged = (alive_row > 0.5) & (rank6 < m)
        mmat = jnp.where((kidx == best) & merged, 1.0, 0.0).astype(jnp.bfloat16)
        ksum = ksum + _onehot_matmul(mmat, incr_parts)
        kcnt = kcnt + jnp.sum(mmat.astype(f32), axis=1, keepdims=True)
        alive_row = alive_row * (1.0 - merged.astype(f32))

    out_kf = ksum / kcnt
    tri6 = idxlt6.astype(f32)
    posi = lax.dot_general(alive_row, tri6, (((1,), (0,)), ((), ())),
                           preferred_element_type=f32)
    pio = lax.broadcasted_iota(jnp.int32, (_IKEEP, _NI), 0).astype(f32)
    perm = jnp.where((posi == pio) & (alive_row > 0.5), 1.0, 0.0).astype(jnp.bfloat16)
    out_incr = _onehot_matmul(perm, incr_parts)
    xc = jnp.concatenate([out_kf, out_incr], axis=0)
    xc = xc * gate_ref[0, 0]

    h = lax.dot_general(xc, w1_ref[...], (((1,), (0,)), ((), ())),
                        precision=_PREC_MLP, preferred_element_type=f32)
    h = h + b1_ref[...]
    h = 0.5 * h * (1.0 + lax.erf(h * 0.7071067811865476))
    out = lax.dot_general(h, w2_ref[...], (((1,), (0,)), ((), ())),
                          precision=_PREC_MLP, preferred_element_type=f32)
    out = out + b2_ref[...]
    out_ref[c] = out


def kernel(x, W1, b1, W2, b2, compress=1, local_num_frames=4):
    assert x.shape[1] == _KF
    n_clips = x.shape[0] // _LNF
    gate = (jnp.asarray(compress, jnp.float32)
            * (jnp.asarray(local_num_frames, jnp.float32)
               / jnp.asarray(local_num_frames, jnp.float32))).reshape(1, 1)
    b1r = b1.reshape(1, -1)
    b2r = b2.reshape(1, -1)

    out = pl.pallas_call(
        _body,
        grid=(n_clips // _CPB,),
        in_specs=[
            pl.BlockSpec((1, 1), lambda i: (0, 0)),
            pl.BlockSpec((_CPB * _LNF, _KF, _D), lambda i: (i, 0, 0)),
            pl.BlockSpec(W1.shape, lambda i: (0, 0)),
            pl.BlockSpec((1, b1.shape[0]), lambda i: (0, 0)),
            pl.BlockSpec(W2.shape, lambda i: (0, 0)),
            pl.BlockSpec((1, b2.shape[0]), lambda i: (0, 0)),
        ],
        out_specs=pl.BlockSpec((_CPB, _KF, W2.shape[1]), lambda i: (i, 0, 0)),
        out_shape=jax.ShapeDtypeStruct((n_clips, _KF, W2.shape[1]),
                                       jnp.float32),
        compiler_params=pltpu.CompilerParams(
            dimension_semantics=("parallel",),
        ),
    )(gate, x, W1, b1r, W2, b2r)
    return out
